# Initial kernel scaffold; baseline (speedup 1.0000x reference)
import jax
import jax.numpy as jnp
from jax.experimental import pallas as pl


def kernel(x, W):
    raise NotImplementedError("write your pallas kernel here")


# speedup vs baseline: 2.8600x; 2.8600x over previous
import functools

import jax
import jax.numpy as jnp
from jax import lax
from jax.experimental import pallas as pl
from jax.experimental.pallas import tpu as pltpu
from jax.experimental.pallas import tpu_sc as plsc

_NBUF = 4


@functools.cache
def _build_sc_kernel(B, L, V, D, G_ROWS):
    info = plsc.get_sparse_core_info()
    NW = info.num_cores * info.num_subcores
    NLANE = info.num_lanes
    B_PER = B // NW
    GIDX = G_ROWS * L
    NGRP = B_PER // G_ROWS
    assert GIDX <= 128 and D % NLANE == 0 and NGRP % _NBUF == 0
    n_vec = D // NLANE

    mesh = plsc.VectorSubcoreMesh(core_axis_name="c", subcore_axis_name="s")

    @functools.partial(
        pl.kernel,
        mesh=mesh,
        out_type=jax.ShapeDtypeStruct((NW, B_PER * D), jnp.float32),
        scratch_types=[
            pltpu.VMEM((NGRP, GIDX), jnp.int32),
            pltpu.VMEM((_NBUF, GIDX, D), jnp.float32),
            pltpu.VMEM((B_PER * D,), jnp.float32),
            [pltpu.SemaphoreType.DMA] * _NBUF,
        ],
        compiler_params=pltpu.CompilerParams(use_tc_tiling_on_sc=False),
    )
    def body(idx_hbm, table_hbm, out_hbm, idx_v, bufs, out_v, sems):
        wid = lax.axis_index("s") * info.num_cores + lax.axis_index("c")
        pltpu.sync_copy(idx_hbm.at[wid], idx_v)

        def start(c, b):
            pltpu.async_copy(table_hbm.at[idx_v.at[c]], bufs.at[b], sems[b])

        def drain(c, b):
            pltpu.make_async_copy(
                table_hbm.at[idx_v.at[c]], bufs.at[b], sems[b]
            ).wait()

        def reduce_group(c, b):
            for k in range(G_ROWS):
                base = k * L
                accs = [bufs[b, base, pl.ds(v * NLANE, NLANE)]
                        for v in range(n_vec)]
                for j in range(1, L):
                    for v in range(n_vec):
                        accs[v] += bufs[b, base + j, pl.ds(v * NLANE, NLANE)]
                off = (c * G_ROWS + k) * D
                for v in range(n_vec):
                    out_v[pl.ds(off + v * NLANE, NLANE)] = (
                        accs[v] * (1.0 / L))

        for b in range(_NBUF):
            start(b, b)

        def loop_body(g, carry):
            for b in range(_NBUF):
                c = g * _NBUF + b
                drain(c, b)
                reduce_group(c, b)

                @pl.when(g < NGRP // _NBUF - 1)
                def _():
                    start(c + _NBUF, b)
            return carry

        lax.fori_loop(0, NGRP // _NBUF, loop_body, 0)
        pltpu.sync_copy(out_v, out_hbm.at[wid])

    return body


def kernel(x, W):
    B, L = x.shape
    V, D = W.shape
    NW = 32
    G_ROWS = 2
    sc = _build_sc_kernel(B, L, V, D, G_ROWS)
    idx = x.astype(jnp.int32).reshape(NW, (B // NW) // G_ROWS, G_ROWS * L)
    out = sc(idx, W)
    return out.reshape(B, D)
